# baseline (speedup 1.0000x reference)
import jax
import jax.numpy as jnp
from jax.experimental import pallas as pl

N = 100000
D = 128
K = 16
BLOCK_ROWS = 16384
P = BLOCK_ROWS // 8
GRID = (N + BLOCK_ROWS - 1) // BLOCK_ROWS

_F32 = jnp.float32
_DN = (((1,), (1,)), ((), ()))


def _body(x_ref, c_ref, o_ref):
    c = c_ref[...]
    cm = -2.0 * c
    ones_kd = jnp.ones((K, D), _F32)
    rows = []
    for j in range(8):
        xj = x_ref[pl.Slice(j, P, 8), :]
        s1 = jax.lax.dot_general(cm, xj, _DN, preferred_element_type=_F32)
        s2 = jax.lax.dot_general(ones_kd, xj * xj, _DN,
                                 preferred_element_type=_F32)
        rows.append(s1 + s2)
    u = jnp.concatenate(rows, axis=0)
    b = jnp.tile(1.0 + jnp.sum(c * c, axis=1, keepdims=True), (8, 1))
    t = jnp.maximum(u + b, 1.0)
    r = 1.0 / t
    i2 = jax.lax.broadcasted_iota(jnp.int32, (128, 128), 0)
    j2 = jax.lax.broadcasted_iota(jnp.int32, (128, 128), 1)
    bd = ((i2 // K) == (j2 // K)).astype(_F32)
    s = jax.lax.dot_general(bd, r, (((1,), (0,)), ((), ())),
                            preferred_element_type=_F32)
    qn = r / s
    eye = (i2 == j2).astype(_F32)
    packed = jax.lax.dot_general(qn, eye, (((0,), (0,)), ((), ())),
                                 preferred_element_type=_F32)
    o_ref[...] = packed


def kernel(x, centers):
    packed = pl.pallas_call(
        _body,
        grid=(GRID,),
        in_specs=[
            pl.BlockSpec((BLOCK_ROWS, D), lambda i: (i, 0)),
            pl.BlockSpec((K, D), lambda i: (0, 0)),
        ],
        out_specs=pl.BlockSpec((P, 128), lambda i: (i, 0)),
        out_shape=jax.ShapeDtypeStruct((N // 8, 128), jnp.float32),
    )(x, centers)
    return packed
